# baseline (speedup 1.0000x reference)
import functools

import jax
import jax.numpy as jnp
from jax import lax
from jax.experimental import pallas as pl
from jax.experimental.pallas import tpu as pltpu
from jax.experimental.pallas import tpu_sc as plsc

B = 16384
N_FIELDS = 26
VOCAB = 100000
EMB_DIM = 16
N_DENSE = 13
N_CH = 8
TEMP = 0.1
KEEP_PROB = 0.8

TOTAL = B * N_FIELDS
NW = 32
PER_W = TOTAL // NW
CHUNK = 3328
N_CHUNKS = PER_W // CHUNK
LANES = 16


def _sc_gather_kernel(ids_hbm, table_hbm, out_hbm, ids_v, idx_v, rows_v, sem):
    nc = 2
    wid = lax.axis_index("s") * nc + lax.axis_index("c")
    base_w = wid * PER_W
    iota = lax.iota(jnp.int32, LANES)

    def chunk_body(ci, _):
        base = base_w + ci * CHUNK
        pltpu.sync_copy(ids_hbm.at[pl.ds(base, CHUNK)], ids_v)

        def vec_body(j, _):
            off = pl.multiple_of(j * LANES, LANES)
            pos = (base + off) + iota
            field = pos % N_FIELDS
            idx_v[pl.ds(off, LANES)] = ids_v[pl.ds(off, LANES)] + field * VOCAB
            return 0

        lax.fori_loop(0, CHUNK // LANES, vec_body, 0)
        pltpu.async_copy(table_hbm.at[idx_v], rows_v, sem).wait()
        pltpu.sync_copy(rows_v, out_hbm.at[pl.ds(base, CHUNK)])
        return 0

    lax.fori_loop(0, N_CHUNKS, chunk_body, 0)


@jax.jit
def _sc_gather(ids_flat, table):
    mesh = plsc.VectorSubcoreMesh(core_axis_name="c", subcore_axis_name="s")
    return pl.kernel(
        _sc_gather_kernel,
        mesh=mesh,
        compiler_params=pltpu.CompilerParams(use_tc_tiling_on_sc=False),
        out_type=jax.ShapeDtypeStruct((TOTAL, EMB_DIM), jnp.float32),
        scratch_types=[
            pltpu.VMEM((CHUNK,), jnp.int32),
            pltpu.VMEM((CHUNK,), jnp.int32),
            pltpu.VMEM((CHUNK, EMB_DIM), jnp.float32),
            pltpu.SemaphoreType.DMA,
        ],
    )(ids_flat, table)


def _tc_autodis_kernel(sparse_ref, dense_ref, meta_ref, w_ref, m_ref, out_ref):
    out_ref[:, : N_FIELDS * EMB_DIM] = sparse_ref[:]
    d = dense_ref[:]
    for n in range(N_DENSE):
        h = d[:, n : n + 1] * w_ref[n : n + 1, :]
        h = jnp.where(h >= 0, h, 0.01 * h)
        m = m_ref[n]
        xb = lax.dot_general(h, m, (((1,), (1,)), ((), ())),
                             preferred_element_type=jnp.float32)
        xb = xb + KEEP_PROB * h
        xb = xb * (1.0 / TEMP)
        mx = jnp.max(xb, axis=1, keepdims=True)
        e = jnp.exp(xb - mx)
        s = jnp.sum(e, axis=1, keepdims=True)
        xh = e / s
        emb = jnp.dot(xh, meta_ref[n],
                      preferred_element_type=jnp.float32)
        col = N_FIELDS * EMB_DIM + n * EMB_DIM
        out_ref[:, col : col + EMB_DIM] = emb


@jax.jit
def _tc_autodis(sparse_out, dense_input, meta_emb, proj_w, proj_m):
    nb = 512
    grid = (B // nb,)
    return pl.pallas_call(
        _tc_autodis_kernel,
        grid=grid,
        in_specs=[
            pl.BlockSpec((nb, N_FIELDS * EMB_DIM), lambda i: (i, 0)),
            pl.BlockSpec((nb, N_DENSE), lambda i: (i, 0)),
            pl.BlockSpec((N_DENSE, N_CH, EMB_DIM), lambda i: (0, 0, 0)),
            pl.BlockSpec((N_DENSE, N_CH), lambda i: (0, 0)),
            pl.BlockSpec((N_DENSE, N_CH, N_CH), lambda i: (0, 0, 0)),
        ],
        out_specs=pl.BlockSpec((nb, N_FIELDS * EMB_DIM + N_DENSE * EMB_DIM),
                               lambda i: (i, 0)),
        out_shape=jax.ShapeDtypeStruct(
            (B, N_FIELDS * EMB_DIM + N_DENSE * EMB_DIM), jnp.float32),
    )(sparse_out, dense_input, meta_emb, proj_w, proj_m)


def kernel(sparse_ids, dense_input, table, meta_emb, proj_w, proj_m):
    ids_flat = sparse_ids.reshape(TOTAL).astype(jnp.int32)
    rows = _sc_gather(ids_flat, table)
    sparse_out = rows.reshape(B, N_FIELDS * EMB_DIM)
    return _tc_autodis(sparse_out, dense_input, meta_emb, proj_w, proj_m)
